# baseline (device time: 72009 ns/iter reference)
import jax
import jax.numpy as jnp
from jax import lax
from jax.experimental import pallas as pl
from jax.experimental.pallas import tpu as pltpu

N_DEV = 4
N_TOK = 2048
D = 512
H = 1024
N_EXP = 16
E_LOCAL = N_EXP // N_DEV
SIDE = 512
QTR = 256


def kernel(x, router_W, route_idx, expert_W, shared_W):
    def body(x_ref, router_ref, idx_ref, ew_ref, sw_ref, out_ref,
             gate_ref, ew_bf, sw_bf, accA_ref, accB_ref,
             a1_s, a1_r, a2_s, a2_r, sideA, a4_r,
             b1_s, b1_r, b2_s, b2_r, sideB, b4_r,
             aS, aR, bS, bR):
        my_pos = lax.axis_index("i")
        left = lax.rem(my_pos + N_DEV - 1, N_DEV)
        right = lax.rem(my_pos + 1, N_DEV)
        p_xor = my_pos ^ 1
        p_mir = 3 - my_pos

        posA = jnp.where(my_pos == 0, 0,
                         jnp.where(my_pos == 1, 2,
                                   jnp.where(my_pos == 2, 3, 1)))
        myA_side = (posA // 2) * SIDE
        othA_side = SIDE - myA_side
        myA_off = (posA % 2) * QTR
        prA_off = QTR - myA_off
        sB = my_pos // 2
        myB_side = 2 * SIDE + sB * SIDE
        othB_side = 2 * SIDE + (1 - sB) * SIDE
        myB_off = lax.rem(my_pos, 2) * QTR
        prB_off = QTR - myB_off

        barrier_sem = pltpu.get_barrier_semaphore()
        for nbr in (left, right):
            pl.semaphore_signal(
                barrier_sem, inc=1,
                device_id=(nbr,), device_id_type=pl.DeviceIdType.MESH,
            )
        pl.semaphore_wait(barrier_sem, 2)

        xv = x_ref[:, :]
        scores = jnp.dot(xv, router_ref[:, :], preferred_element_type=jnp.float32)
        s_max = jnp.max(scores, axis=-1, keepdims=True)
        probs = jnp.exp(scores - s_max)
        probs = probs / jnp.sum(probs, axis=-1, keepdims=True)
        route = idx_ref[:, :]
        iota = lax.broadcasted_iota(jnp.int32, (N_TOK, N_EXP), 1)
        onehot = (iota == route).astype(jnp.float32)
        p_tok = jnp.sum(probs * onehot, axis=-1, keepdims=True)
        for le in range(E_LOCAL):
            ge = my_pos * E_LOCAL + le
            gate_ref[:, le:le + 1] = jnp.where(route == ge, p_tok, 0.0)

        for le in range(E_LOCAL):
            ew_bf[le] = ew_ref[le].astype(jnp.bfloat16)
        sw_bf[:, :] = sw_ref[:, :].astype(jnp.bfloat16)

        def expert_part(row0):
            x_c = x_ref[pl.ds(row0, SIDE), :].astype(jnp.bfloat16)
            acc = jnp.zeros((SIDE, H), jnp.float32)
            for le in range(E_LOCAL):
                y = jnp.dot(x_c, ew_bf[le], preferred_element_type=jnp.float32)
                acc = acc + gate_ref[pl.ds(row0, SIDE), le:le + 1] * y
            return acc

        def exch(src, dst, sem_arr_s, sem_arr_r, k, partner):
            return pltpu.make_async_remote_copy(
                src_ref=src, dst_ref=dst,
                send_sem=sem_arr_s.at[k], recv_sem=sem_arr_r.at[k],
                device_id=(partner,), device_id_type=pl.DeviceIdType.MESH,
            )

        a1_s[:, :] = expert_part(othA_side).astype(jnp.bfloat16)
        A1 = exch(a1_s, a1_r, aS, aR, 0, p_xor)
        A1.start()
        b1_s[:, :] = expert_part(othB_side).astype(jnp.bfloat16)
        B1 = exch(b1_s, b1_r, bS, bR, 0, p_mir)
        B1.start()

        accA_ref[:, :] = expert_part(myA_side)
        accB_ref[:, :] = expert_part(myB_side)
        shA = jnp.dot(
            x_ref[pl.ds(myA_side + myA_off, QTR), :].astype(jnp.bfloat16),
            sw_bf[:, :], preferred_element_type=jnp.float32)
        shB = jnp.dot(
            x_ref[pl.ds(myB_side + myB_off, QTR), :].astype(jnp.bfloat16),
            sw_bf[:, :], preferred_element_type=jnp.float32)

        A1.wait()
        accA_ref[:, :] = accA_ref[:, :] + a1_r[:, :].astype(jnp.float32)
        a2_s[:, :] = accA_ref[pl.ds(prA_off, QTR), :].astype(jnp.bfloat16)
        A2 = exch(a2_s, a2_r, aS, aR, 1, p_mir)
        A2.start()
        B1.wait()
        accB_ref[:, :] = accB_ref[:, :] + b1_r[:, :].astype(jnp.float32)
        b2_s[:, :] = accB_ref[pl.ds(prB_off, QTR), :].astype(jnp.bfloat16)
        B2 = exch(b2_s, b2_r, bS, bR, 1, p_xor)
        B2.start()

        A2.wait()
        finA = (accA_ref[pl.ds(myA_off, QTR), :]
                + a2_r[:, :].astype(jnp.float32) + shA)
        out_ref[pl.ds(myA_side + myA_off, QTR), :] = finA
        sideA[pl.ds(myA_off, QTR), :] = finA.astype(jnp.bfloat16)
        A3 = exch(sideA.at[pl.ds(myA_off, QTR), :],
                  sideA.at[pl.ds(myA_off, QTR), :], aS, aR, 2, p_mir)
        A3.start()
        B2.wait()
        finB = (accB_ref[pl.ds(myB_off, QTR), :]
                + b2_r[:, :].astype(jnp.float32) + shB)
        out_ref[pl.ds(myB_side + myB_off, QTR), :] = finB
        sideB[pl.ds(myB_off, QTR), :] = finB.astype(jnp.bfloat16)
        B3 = exch(sideB.at[pl.ds(myB_off, QTR), :],
                  sideB.at[pl.ds(myB_off, QTR), :], bS, bR, 2, p_xor)
        B3.start()

        A3.wait()
        A4 = exch(sideA, a4_r, aS, aR, 3, p_xor)
        A4.start()
        out_ref[pl.ds(myA_side + prA_off, QTR), :] = (
            sideA[pl.ds(prA_off, QTR), :].astype(jnp.float32))
        B3.wait()
        B4 = exch(sideB, b4_r, bS, bR, 3, p_mir)
        B4.start()
        out_ref[pl.ds(myB_side + prB_off, QTR), :] = (
            sideB[pl.ds(prB_off, QTR), :].astype(jnp.float32))
        A4.wait()
        out_ref[pl.ds(othA_side, SIDE), :] = a4_r[:, :].astype(jnp.float32)
        B4.wait()
        out_ref[pl.ds(othB_side, SIDE), :] = b4_r[:, :].astype(jnp.float32)

    dma4 = pltpu.SemaphoreType.DMA((4,))
    bf = jnp.bfloat16
    return pl.pallas_call(
        body,
        out_shape=jax.ShapeDtypeStruct((N_TOK, H), jnp.float32),
        in_specs=[
            pl.BlockSpec(memory_space=pltpu.VMEM),
            pl.BlockSpec(memory_space=pltpu.VMEM),
            pl.BlockSpec(memory_space=pltpu.VMEM),
            pl.BlockSpec(memory_space=pltpu.VMEM),
            pl.BlockSpec(memory_space=pltpu.VMEM),
        ],
        out_specs=pl.BlockSpec(memory_space=pltpu.VMEM),
        scratch_shapes=[
            pltpu.VMEM((N_TOK, E_LOCAL), jnp.float32),
            pltpu.VMEM((E_LOCAL, D, H), bf),
            pltpu.VMEM((D, H), bf),
            pltpu.VMEM((SIDE, H), jnp.float32),
            pltpu.VMEM((SIDE, H), jnp.float32),
            pltpu.VMEM((SIDE, H), bf),
            pltpu.VMEM((SIDE, H), bf),
            pltpu.VMEM((QTR, H), bf),
            pltpu.VMEM((QTR, H), bf),
            pltpu.VMEM((SIDE, H), bf),
            pltpu.VMEM((SIDE, H), bf),
            pltpu.VMEM((SIDE, H), bf),
            pltpu.VMEM((SIDE, H), bf),
            pltpu.VMEM((QTR, H), bf),
            pltpu.VMEM((QTR, H), bf),
            pltpu.VMEM((SIDE, H), bf),
            pltpu.VMEM((SIDE, H), bf),
            dma4, dma4,
            dma4, dma4,
        ],
        compiler_params=pltpu.CompilerParams(
            collective_id=0, vmem_limit_bytes=100 * 1024 * 1024,
        ),
    )(x, router_W, route_idx, expert_W, shared_W)


# device time: 71868 ns/iter; 1.0020x vs baseline; 1.0020x over previous
import jax
import jax.numpy as jnp
from jax import lax
from jax.experimental import pallas as pl
from jax.experimental.pallas import tpu as pltpu

N_DEV = 4
N_TOK = 2048
D = 512
H = 1024
N_EXP = 16
E_LOCAL = N_EXP // N_DEV
SIDE = 512
QTR = 256


def kernel(x, router_W, route_idx, expert_W, shared_W):
    def body(x_ref, router_ref, idx_ref, ew_ref, sw_ref, out_ref,
             gate_ref, ew_bf, sw_bf, accA_ref, accB_ref,
             a1_s, a1_r, a2_s, a2_r, sideA, a4_r,
             b1_s, b1_r, b2_s, b2_r, sideB, b4_r,
             aS, aR, bS, bR):
        my_pos = lax.axis_index("i")
        left = lax.rem(my_pos + N_DEV - 1, N_DEV)
        right = lax.rem(my_pos + 1, N_DEV)
        p_xor = my_pos ^ 1
        p_mir = 3 - my_pos

        posA = jnp.where(my_pos == 0, 0,
                         jnp.where(my_pos == 1, 2,
                                   jnp.where(my_pos == 2, 3, 1)))
        myA_side = (posA // 2) * SIDE
        othA_side = SIDE - myA_side
        myA_off = (posA % 2) * QTR
        prA_off = QTR - myA_off
        sB = my_pos // 2
        myB_side = 2 * SIDE + sB * SIDE
        othB_side = 2 * SIDE + (1 - sB) * SIDE
        myB_off = lax.rem(my_pos, 2) * QTR
        prB_off = QTR - myB_off

        barrier_sem = pltpu.get_barrier_semaphore()
        for nbr in (left, right):
            pl.semaphore_signal(
                barrier_sem, inc=1,
                device_id=(nbr,), device_id_type=pl.DeviceIdType.MESH,
            )
        pl.semaphore_wait(barrier_sem, 2)

        xv = x_ref[:, :]
        scores = jnp.dot(xv, router_ref[:, :], preferred_element_type=jnp.float32)
        s_max = jnp.max(scores, axis=-1, keepdims=True)
        probs = jnp.exp(scores - s_max)
        probs = probs / jnp.sum(probs, axis=-1, keepdims=True)
        route = idx_ref[:, :]
        iota = lax.broadcasted_iota(jnp.int32, (N_TOK, N_EXP), 1)
        onehot = (iota == route).astype(jnp.float32)
        p_tok = jnp.sum(probs * onehot, axis=-1, keepdims=True)
        for le in range(E_LOCAL):
            ge = my_pos * E_LOCAL + le
            gate_ref[:, le:le + 1] = jnp.where(route == ge, p_tok, 0.0)

        for le in range(E_LOCAL):
            ew_bf[le] = ew_ref[le].astype(jnp.bfloat16)
        sw_bf[:, :] = sw_ref[:, :].astype(jnp.bfloat16)

        def expert_part(row0):
            x_c = x_ref[pl.ds(row0, SIDE), :].astype(jnp.bfloat16)
            acc = jnp.zeros((SIDE, H), jnp.float32)
            for le in range(E_LOCAL):
                y = jnp.dot(x_c, ew_bf[le], preferred_element_type=jnp.float32)
                acc = acc + gate_ref[pl.ds(row0, SIDE), le:le + 1] * y
            return acc

        def exch(src, dst, sem_arr_s, sem_arr_r, k, partner):
            return pltpu.make_async_remote_copy(
                src_ref=src, dst_ref=dst,
                send_sem=sem_arr_s.at[k], recv_sem=sem_arr_r.at[k],
                device_id=(partner,), device_id_type=pl.DeviceIdType.MESH,
            )

        a1_s[:, :] = expert_part(othA_side).astype(jnp.bfloat16)
        A1 = exch(a1_s, a1_r, aS, aR, 0, p_xor)
        A1.start()
        b1_s[:, :] = expert_part(othB_side).astype(jnp.bfloat16)
        B1 = exch(b1_s, b1_r, bS, bR, 0, p_mir)
        B1.start()

        accA_ref[:, :] = expert_part(myA_side)
        accB_ref[:, :] = expert_part(myB_side)
        shA = jnp.dot(
            x_ref[pl.ds(myA_side + myA_off, QTR), :].astype(jnp.bfloat16),
            sw_bf[:, :], preferred_element_type=jnp.float32)
        shB = jnp.dot(
            x_ref[pl.ds(myB_side + myB_off, QTR), :].astype(jnp.bfloat16),
            sw_bf[:, :], preferred_element_type=jnp.float32)

        A1.wait()
        a2_s[:, :] = (accA_ref[pl.ds(prA_off, QTR), :]
                      + a1_r[pl.ds(prA_off, QTR), :].astype(jnp.float32)
                      ).astype(jnp.bfloat16)
        A2 = exch(a2_s, a2_r, aS, aR, 1, p_mir)
        A2.start()
        B1.wait()
        b2_s[:, :] = (accB_ref[pl.ds(prB_off, QTR), :]
                      + b1_r[pl.ds(prB_off, QTR), :].astype(jnp.float32)
                      ).astype(jnp.bfloat16)
        B2 = exch(b2_s, b2_r, bS, bR, 1, p_xor)
        B2.start()

        preA = (accA_ref[pl.ds(myA_off, QTR), :]
                + a1_r[pl.ds(myA_off, QTR), :].astype(jnp.float32) + shA)
        preB = (accB_ref[pl.ds(myB_off, QTR), :]
                + b1_r[pl.ds(myB_off, QTR), :].astype(jnp.float32) + shB)

        A2.wait()
        finA = preA + a2_r[:, :].astype(jnp.float32)
        sideA[pl.ds(myA_off, QTR), :] = finA.astype(jnp.bfloat16)
        A3 = exch(sideA.at[pl.ds(myA_off, QTR), :],
                  sideA.at[pl.ds(myA_off, QTR), :], aS, aR, 2, p_mir)
        A3.start()
        out_ref[pl.ds(myA_side + myA_off, QTR), :] = finA
        B2.wait()
        finB = preB + b2_r[:, :].astype(jnp.float32)
        sideB[pl.ds(myB_off, QTR), :] = finB.astype(jnp.bfloat16)
        B3 = exch(sideB.at[pl.ds(myB_off, QTR), :],
                  sideB.at[pl.ds(myB_off, QTR), :], bS, bR, 2, p_xor)
        B3.start()
        out_ref[pl.ds(myB_side + myB_off, QTR), :] = finB

        A3.wait()
        A4 = exch(sideA, a4_r, aS, aR, 3, p_xor)
        A4.start()
        out_ref[pl.ds(myA_side + prA_off, QTR), :] = (
            sideA[pl.ds(prA_off, QTR), :].astype(jnp.float32))
        B3.wait()
        B4 = exch(sideB, b4_r, bS, bR, 3, p_mir)
        B4.start()
        out_ref[pl.ds(myB_side + prB_off, QTR), :] = (
            sideB[pl.ds(prB_off, QTR), :].astype(jnp.float32))
        A4.wait()
        out_ref[pl.ds(othA_side, SIDE), :] = a4_r[:, :].astype(jnp.float32)
        B4.wait()
        out_ref[pl.ds(othB_side, SIDE), :] = b4_r[:, :].astype(jnp.float32)

    dma4 = pltpu.SemaphoreType.DMA((4,))
    bf = jnp.bfloat16
    return pl.pallas_call(
        body,
        out_shape=jax.ShapeDtypeStruct((N_TOK, H), jnp.float32),
        in_specs=[
            pl.BlockSpec(memory_space=pltpu.VMEM),
            pl.BlockSpec(memory_space=pltpu.VMEM),
            pl.BlockSpec(memory_space=pltpu.VMEM),
            pl.BlockSpec(memory_space=pltpu.VMEM),
            pl.BlockSpec(memory_space=pltpu.VMEM),
        ],
        out_specs=pl.BlockSpec(memory_space=pltpu.VMEM),
        scratch_shapes=[
            pltpu.VMEM((N_TOK, E_LOCAL), jnp.float32),
            pltpu.VMEM((E_LOCAL, D, H), bf),
            pltpu.VMEM((D, H), bf),
            pltpu.VMEM((SIDE, H), jnp.float32),
            pltpu.VMEM((SIDE, H), jnp.float32),
            pltpu.VMEM((SIDE, H), bf),
            pltpu.VMEM((SIDE, H), bf),
            pltpu.VMEM((QTR, H), bf),
            pltpu.VMEM((QTR, H), bf),
            pltpu.VMEM((SIDE, H), bf),
            pltpu.VMEM((SIDE, H), bf),
            pltpu.VMEM((SIDE, H), bf),
            pltpu.VMEM((SIDE, H), bf),
            pltpu.VMEM((QTR, H), bf),
            pltpu.VMEM((QTR, H), bf),
            pltpu.VMEM((SIDE, H), bf),
            pltpu.VMEM((SIDE, H), bf),
            dma4, dma4,
            dma4, dma4,
        ],
        compiler_params=pltpu.CompilerParams(
            collective_id=0, vmem_limit_bytes=100 * 1024 * 1024,
        ),
    )(x, router_W, route_idx, expert_W, shared_W)


# device time: 68847 ns/iter; 1.0459x vs baseline; 1.0439x over previous
import jax
import jax.numpy as jnp
from jax import lax
from jax.experimental import pallas as pl
from jax.experimental.pallas import tpu as pltpu

N_DEV = 4
N_TOK = 2048
D = 512
H = 1024
N_EXP = 16
E_LOCAL = N_EXP // N_DEV
SIDE = 512
QTR = 256


def kernel(x, router_W, route_idx, expert_W, shared_W):
    def body(x_ref, router_ref, idx_ref, ew_ref, sw_ref, out_ref,
             gate_ref, ew_bf, sw_bf,
             a1_s, a1_r, a2_s, a2_r, sideA, a4_r,
             b1_s, b1_r, b2_s, b2_r, sideB, b4_r,
             aS, aR, bS, bR):
        my_pos = lax.axis_index("i")
        left = lax.rem(my_pos + N_DEV - 1, N_DEV)
        right = lax.rem(my_pos + 1, N_DEV)
        p_xor = my_pos ^ 1
        p_mir = 3 - my_pos

        posA = jnp.where(my_pos == 0, 0,
                         jnp.where(my_pos == 1, 2,
                                   jnp.where(my_pos == 2, 3, 1)))
        myA_side = (posA // 2) * SIDE
        othA_side = SIDE - myA_side
        myA_off = (posA % 2) * QTR
        prA_off = QTR - myA_off
        sB = my_pos // 2
        myB_side = 2 * SIDE + sB * SIDE
        othB_side = 2 * SIDE + (1 - sB) * SIDE
        myB_off = lax.rem(my_pos, 2) * QTR
        prB_off = QTR - myB_off

        barrier_sem = pltpu.get_barrier_semaphore()
        for nbr in (left, right):
            pl.semaphore_signal(
                barrier_sem, inc=1,
                device_id=(nbr,), device_id_type=pl.DeviceIdType.MESH,
            )
        pl.semaphore_wait(barrier_sem, 2)

        xv = x_ref[:, :]
        scores = jnp.dot(xv, router_ref[:, :], preferred_element_type=jnp.float32)
        s_max = jnp.max(scores, axis=-1, keepdims=True)
        probs = jnp.exp(scores - s_max)
        probs = probs / jnp.sum(probs, axis=-1, keepdims=True)
        route = idx_ref[:, :]
        iota = lax.broadcasted_iota(jnp.int32, (N_TOK, N_EXP), 1)
        onehot = (iota == route).astype(jnp.float32)
        p_tok = jnp.sum(probs * onehot, axis=-1, keepdims=True)
        for le in range(E_LOCAL):
            ge = my_pos * E_LOCAL + le
            gate_ref[:, le:le + 1] = jnp.where(route == ge, p_tok, 0.0)

        for le in range(E_LOCAL):
            ew_bf[le] = ew_ref[le].astype(jnp.bfloat16)
        sw_bf[:, :] = sw_ref[:, :].astype(jnp.bfloat16)

        def part(row0):
            x_c = x_ref[pl.ds(row0, QTR), :].astype(jnp.bfloat16)
            acc = jnp.zeros((QTR, H), jnp.float32)
            for le in range(E_LOCAL):
                y = jnp.dot(x_c, ew_bf[le], preferred_element_type=jnp.float32)
                acc = acc + gate_ref[pl.ds(row0, QTR), le:le + 1] * y
            return acc

        def exch(src, dst, sems_s, sems_r, k, partner):
            return pltpu.make_async_remote_copy(
                src_ref=src, dst_ref=dst,
                send_sem=sems_s.at[k], recv_sem=sems_r.at[k],
                device_id=(partner,), device_id_type=pl.DeviceIdType.MESH,
            )

        f32 = jnp.float32
        bf = jnp.bfloat16

        a1_s[pl.ds(prA_off, QTR), :] = part(othA_side + prA_off).astype(bf)
        A1a = exch(a1_s.at[pl.ds(prA_off, QTR), :],
                   a1_r.at[pl.ds(prA_off, QTR), :], aS, aR, 0, p_xor)
        A1a.start()
        b1_s[pl.ds(myB_off, QTR), :] = part(othB_side + myB_off).astype(bf)
        B1a = exch(b1_s.at[pl.ds(myB_off, QTR), :],
                   b1_r.at[pl.ds(myB_off, QTR), :], bS, bR, 0, p_mir)
        B1a.start()
        a1_s[pl.ds(myA_off, QTR), :] = part(othA_side + myA_off).astype(bf)
        A1b = exch(a1_s.at[pl.ds(myA_off, QTR), :],
                   a1_r.at[pl.ds(myA_off, QTR), :], aS, aR, 1, p_xor)
        A1b.start()
        b1_s[pl.ds(prB_off, QTR), :] = part(othB_side + prB_off).astype(bf)
        B1b = exch(b1_s.at[pl.ds(prB_off, QTR), :],
                   b1_r.at[pl.ds(prB_off, QTR), :], bS, bR, 1, p_mir)
        B1b.start()

        kA_pr = part(myA_side + prA_off)
        kB_pr = part(myB_side + prB_off)

        A1a.wait()
        a2_s[:, :] = (kA_pr + a1_r[pl.ds(prA_off, QTR), :].astype(f32)
                      ).astype(bf)
        A2 = exch(a2_s, a2_r, aS, aR, 2, p_mir)
        A2.start()
        B1a.wait()
        b2_s[:, :] = (kB_pr + b1_r[pl.ds(prB_off, QTR), :].astype(f32)
                      ).astype(bf)
        B2 = exch(b2_s, b2_r, bS, bR, 2, p_xor)
        B2.start()

        kA_my = part(myA_side + myA_off)
        kB_my = part(myB_side + myB_off)
        shA = jnp.dot(x_ref[pl.ds(myA_side + myA_off, QTR), :].astype(bf),
                      sw_bf[:, :], preferred_element_type=f32)
        shB = jnp.dot(x_ref[pl.ds(myB_side + myB_off, QTR), :].astype(bf),
                      sw_bf[:, :], preferred_element_type=f32)
        A1b.wait()
        preA = kA_my + a1_r[pl.ds(myA_off, QTR), :].astype(f32) + shA
        B1b.wait()
        preB = kB_my + b1_r[pl.ds(myB_off, QTR), :].astype(f32) + shB

        A2.wait()
        finA = preA + a2_r[:, :].astype(f32)
        sideA[pl.ds(myA_off, QTR), :] = finA.astype(bf)
        A3 = exch(sideA.at[pl.ds(myA_off, QTR), :],
                  sideA.at[pl.ds(myA_off, QTR), :], aS, aR, 3, p_mir)
        A3.start()
        A4a = exch(sideA.at[pl.ds(myA_off, QTR), :],
                   a4_r.at[pl.ds(myA_off, QTR), :], aS, aR, 4, p_xor)
        A4a.start()
        out_ref[pl.ds(myA_side + myA_off, QTR), :] = finA
        B2.wait()
        finB = preB + b2_r[:, :].astype(f32)
        sideB[pl.ds(myB_off, QTR), :] = finB.astype(bf)
        B3 = exch(sideB.at[pl.ds(myB_off, QTR), :],
                  sideB.at[pl.ds(myB_off, QTR), :], bS, bR, 3, p_xor)
        B3.start()
        B4a = exch(sideB.at[pl.ds(myB_off, QTR), :],
                   b4_r.at[pl.ds(myB_off, QTR), :], bS, bR, 4, p_mir)
        B4a.start()
        out_ref[pl.ds(myB_side + myB_off, QTR), :] = finB

        A3.wait()
        A4b = exch(sideA.at[pl.ds(prA_off, QTR), :],
                   a4_r.at[pl.ds(prA_off, QTR), :], aS, aR, 5, p_xor)
        A4b.start()
        out_ref[pl.ds(myA_side + prA_off, QTR), :] = (
            sideA[pl.ds(prA_off, QTR), :].astype(f32))
        B3.wait()
        B4b = exch(sideB.at[pl.ds(prB_off, QTR), :],
                   b4_r.at[pl.ds(prB_off, QTR), :], bS, bR, 5, p_mir)
        B4b.start()
        out_ref[pl.ds(myB_side + prB_off, QTR), :] = (
            sideB[pl.ds(prB_off, QTR), :].astype(f32))

        A4a.wait()
        out_ref[pl.ds(othA_side + myA_off, QTR), :] = (
            a4_r[pl.ds(myA_off, QTR), :].astype(f32))
        B4a.wait()
        out_ref[pl.ds(othB_side + myB_off, QTR), :] = (
            b4_r[pl.ds(myB_off, QTR), :].astype(f32))
        A4b.wait()
        out_ref[pl.ds(othA_side + prA_off, QTR), :] = (
            a4_r[pl.ds(prA_off, QTR), :].astype(f32))
        B4b.wait()
        out_ref[pl.ds(othB_side + prB_off, QTR), :] = (
            b4_r[pl.ds(prB_off, QTR), :].astype(f32))

    dma6 = pltpu.SemaphoreType.DMA((6,))
    bf = jnp.bfloat16
    return pl.pallas_call(
        body,
        out_shape=jax.ShapeDtypeStruct((N_TOK, H), jnp.float32),
        in_specs=[pl.BlockSpec(memory_space=pltpu.VMEM)] * 5,
        out_specs=pl.BlockSpec(memory_space=pltpu.VMEM),
        scratch_shapes=[
            pltpu.VMEM((N_TOK, E_LOCAL), jnp.float32),
            pltpu.VMEM((E_LOCAL, D, H), bf),
            pltpu.VMEM((D, H), bf),
            pltpu.VMEM((SIDE, H), bf),
            pltpu.VMEM((SIDE, H), bf),
            pltpu.VMEM((QTR, H), bf),
            pltpu.VMEM((QTR, H), bf),
            pltpu.VMEM((SIDE, H), bf),
            pltpu.VMEM((SIDE, H), bf),
            pltpu.VMEM((SIDE, H), bf),
            pltpu.VMEM((SIDE, H), bf),
            pltpu.VMEM((QTR, H), bf),
            pltpu.VMEM((QTR, H), bf),
            pltpu.VMEM((SIDE, H), bf),
            pltpu.VMEM((SIDE, H), bf),
            dma6, dma6,
            dma6, dma6,
        ],
        compiler_params=pltpu.CompilerParams(
            collective_id=0, vmem_limit_bytes=100 * 1024 * 1024,
        ),
    )(x, router_W, route_idx, expert_W, shared_W)


# device time: 68766 ns/iter; 1.0472x vs baseline; 1.0012x over previous
import jax
import jax.numpy as jnp
from jax import lax
from jax.experimental import pallas as pl
from jax.experimental.pallas import tpu as pltpu

N_DEV = 4
N_TOK = 2048
D = 512
H = 1024
N_EXP = 16
E_LOCAL = N_EXP // N_DEV
SIDE = 512
QTR = 256


def kernel(x, router_W, route_idx, expert_W, shared_W):
    def body(x_ref, router_ref, idx_ref, ew_ref, sw_ref, out_ref,
             gate_ref, ew_bf, sw_bf,
             a1_s, a1_r, a2_s, a2_r, sideA, a4_r,
             b1_s, b1_r, b2_s, b2_r, sideB, b4_r,
             aS, aR, bS, bR):
        my_pos = lax.axis_index("i")
        left = lax.rem(my_pos + N_DEV - 1, N_DEV)
        right = lax.rem(my_pos + 1, N_DEV)
        p_xor = my_pos ^ 1
        p_mir = 3 - my_pos

        posA = jnp.where(my_pos == 0, 0,
                         jnp.where(my_pos == 1, 2,
                                   jnp.where(my_pos == 2, 3, 1)))
        myA_side = (posA // 2) * SIDE
        othA_side = SIDE - myA_side
        myA_off = (posA % 2) * QTR
        prA_off = QTR - myA_off
        sB = my_pos // 2
        myB_side = 2 * SIDE + sB * SIDE
        othB_side = 2 * SIDE + (1 - sB) * SIDE
        myB_off = lax.rem(my_pos, 2) * QTR
        prB_off = QTR - myB_off

        barrier_sem = pltpu.get_barrier_semaphore()
        for nbr in (left, right):
            pl.semaphore_signal(
                barrier_sem, inc=1,
                device_id=(nbr,), device_id_type=pl.DeviceIdType.MESH,
            )
        pl.semaphore_wait(barrier_sem, 2)

        xv = x_ref[:, :]
        scores = jnp.dot(xv, router_ref[:, :], preferred_element_type=jnp.float32)
        s_max = jnp.max(scores, axis=-1, keepdims=True)
        probs = jnp.exp(scores - s_max)
        probs = probs / jnp.sum(probs, axis=-1, keepdims=True)
        route = idx_ref[:, :]
        iota = lax.broadcasted_iota(jnp.int32, (N_TOK, N_EXP), 1)
        onehot = (iota == route).astype(jnp.float32)
        p_tok = jnp.sum(probs * onehot, axis=-1, keepdims=True)
        for le in range(E_LOCAL):
            ge = my_pos * E_LOCAL + le
            gate_ref[:, le:le + 1] = jnp.where(route == ge, p_tok, 0.0)

        for le in range(E_LOCAL):
            ew_bf[le] = ew_ref[le].astype(jnp.bfloat16)
        sw_bf[:, :] = sw_ref[:, :].astype(jnp.bfloat16)

        def part(row0):
            x_c = x_ref[pl.ds(row0, QTR), :].astype(jnp.bfloat16)
            acc = jnp.zeros((QTR, H), jnp.float32)
            for le in range(E_LOCAL):
                y = jnp.dot(x_c, ew_bf[le], preferred_element_type=jnp.float32)
                acc = acc + gate_ref[pl.ds(row0, QTR), le:le + 1] * y
            return acc

        def exch(src, dst, sems_s, sems_r, k, partner):
            return pltpu.make_async_remote_copy(
                src_ref=src, dst_ref=dst,
                send_sem=sems_s.at[k], recv_sem=sems_r.at[k],
                device_id=(partner,), device_id_type=pl.DeviceIdType.MESH,
            )

        f32 = jnp.float32
        bf = jnp.bfloat16

        a1_s[pl.ds(prA_off, QTR), :] = part(othA_side + prA_off).astype(bf)
        A1a = exch(a1_s.at[pl.ds(prA_off, QTR), :],
                   a1_r.at[pl.ds(prA_off, QTR), :], aS, aR, 0, p_xor)
        A1a.start()
        b1_s[pl.ds(myB_off, QTR), :] = part(othB_side + myB_off).astype(bf)
        B1a = exch(b1_s.at[pl.ds(myB_off, QTR), :],
                   b1_r.at[pl.ds(myB_off, QTR), :], bS, bR, 0, p_mir)
        B1a.start()
        a1_s[pl.ds(myA_off, QTR), :] = part(othA_side + myA_off).astype(bf)
        A1b = exch(a1_s.at[pl.ds(myA_off, QTR), :],
                   a1_r.at[pl.ds(myA_off, QTR), :], aS, aR, 1, p_xor)
        A1b.start()
        b1_s[pl.ds(prB_off, QTR), :] = part(othB_side + prB_off).astype(bf)
        B1b = exch(b1_s.at[pl.ds(prB_off, QTR), :],
                   b1_r.at[pl.ds(prB_off, QTR), :], bS, bR, 1, p_mir)
        B1b.start()

        kA_pr = part(myA_side + prA_off)
        kB_pr = part(myB_side + prB_off)

        A1a.wait()
        a2_s[:, :] = (kA_pr + a1_r[pl.ds(prA_off, QTR), :].astype(f32)
                      ).astype(bf)
        A2 = exch(a2_s, a2_r, aS, aR, 2, p_mir)
        A2.start()
        B1a.wait()
        b2_s[:, :] = (kB_pr + b1_r[pl.ds(prB_off, QTR), :].astype(f32)
                      ).astype(bf)
        B2 = exch(b2_s, b2_r, bS, bR, 2, p_xor)
        B2.start()

        kA_my = part(myA_side + myA_off)
        kB_my = part(myB_side + myB_off)
        shA = jnp.dot(x_ref[pl.ds(myA_side + myA_off, QTR), :].astype(bf),
                      sw_bf[:, :], preferred_element_type=f32)
        shB = jnp.dot(x_ref[pl.ds(myB_side + myB_off, QTR), :].astype(bf),
                      sw_bf[:, :], preferred_element_type=f32)
        A1b.wait()
        preA = kA_my + a1_r[pl.ds(myA_off, QTR), :].astype(f32) + shA
        B1b.wait()
        preB = kB_my + b1_r[pl.ds(myB_off, QTR), :].astype(f32) + shB

        A2.wait()
        finA = preA + a2_r[:, :].astype(f32)
        sideA[pl.ds(myA_off, QTR), :] = finA.astype(bf)
        A3 = exch(sideA.at[pl.ds(myA_off, QTR), :],
                  sideA.at[pl.ds(myA_off, QTR), :], aS, aR, 3, p_mir)
        A3.start()
        A4a = exch(sideA.at[pl.ds(myA_off, QTR), :],
                   a4_r.at[pl.ds(myA_off, QTR), :], aS, aR, 4, p_xor)
        A4a.start()
        out_ref[pl.ds(myA_side + myA_off, QTR), :] = finA
        B2.wait()
        finB = preB + b2_r[:, :].astype(f32)
        sideB[pl.ds(myB_off, QTR), :] = finB.astype(bf)
        B3 = exch(sideB.at[pl.ds(myB_off, QTR), :],
                  sideB.at[pl.ds(myB_off, QTR), :], bS, bR, 3, p_xor)
        B3.start()
        B4a = exch(sideB.at[pl.ds(myB_off, QTR), :],
                   b4_r.at[pl.ds(myB_off, QTR), :], bS, bR, 4, p_mir)
        B4a.start()
        out_ref[pl.ds(myB_side + myB_off, QTR), :] = finB

        A3.wait()
        A4b = exch(sideA.at[pl.ds(prA_off, QTR), :],
                   a4_r.at[pl.ds(prA_off, QTR), :], aS, aR, 5, p_xor)
        A4b.start()
        out_ref[pl.ds(myA_side + prA_off, QTR), :] = (
            sideA[pl.ds(prA_off, QTR), :].astype(f32))
        B3.wait()
        B4b = exch(sideB.at[pl.ds(prB_off, QTR), :],
                   b4_r.at[pl.ds(prB_off, QTR), :], bS, bR, 5, p_mir)
        B4b.start()
        out_ref[pl.ds(myB_side + prB_off, QTR), :] = (
            sideB[pl.ds(prB_off, QTR), :].astype(f32))

        A4a.wait()
        out_ref[pl.ds(othA_side + myA_off, QTR), :] = (
            a4_r[pl.ds(myA_off, QTR), :].astype(f32))
        B4a.wait()
        out_ref[pl.ds(othB_side + prB_off, QTR), :] = (
            b4_r[pl.ds(prB_off, QTR), :].astype(f32))
        A4b.wait()
        out_ref[pl.ds(othA_side + prA_off, QTR), :] = (
            a4_r[pl.ds(prA_off, QTR), :].astype(f32))
        B4b.wait()
        out_ref[pl.ds(othB_side + myB_off, QTR), :] = (
            b4_r[pl.ds(myB_off, QTR), :].astype(f32))

    dma6 = pltpu.SemaphoreType.DMA((6,))
    bf = jnp.bfloat16
    return pl.pallas_call(
        body,
        out_shape=jax.ShapeDtypeStruct((N_TOK, H), jnp.float32),
        in_specs=[pl.BlockSpec(memory_space=pltpu.VMEM)] * 5,
        out_specs=pl.BlockSpec(memory_space=pltpu.VMEM),
        scratch_shapes=[
            pltpu.VMEM((N_TOK, E_LOCAL), jnp.float32),
            pltpu.VMEM((E_LOCAL, D, H), bf),
            pltpu.VMEM((D, H), bf),
            pltpu.VMEM((SIDE, H), bf),
            pltpu.VMEM((SIDE, H), bf),
            pltpu.VMEM((QTR, H), bf),
            pltpu.VMEM((QTR, H), bf),
            pltpu.VMEM((SIDE, H), bf),
            pltpu.VMEM((SIDE, H), bf),
            pltpu.VMEM((SIDE, H), bf),
            pltpu.VMEM((SIDE, H), bf),
            pltpu.VMEM((QTR, H), bf),
            pltpu.VMEM((QTR, H), bf),
            pltpu.VMEM((SIDE, H), bf),
            pltpu.VMEM((SIDE, H), bf),
            dma6, dma6,
            dma6, dma6,
        ],
        compiler_params=pltpu.CompilerParams(
            collective_id=0, vmem_limit_bytes=100 * 1024 * 1024,
        ),
    )(x, router_W, route_idx, expert_W, shared_W)


# device time: 64960 ns/iter; 1.1085x vs baseline; 1.0586x over previous
import jax
import jax.numpy as jnp
from jax import lax
from jax.experimental import pallas as pl
from jax.experimental.pallas import tpu as pltpu

N_DEV = 4
N_TOK = 2048
D = 512
H = 1024
HCOL = H // 2
N_EXP = 16
E_LOCAL = N_EXP // N_DEV
CHUNK = N_TOK // N_DEV
HALF = CHUNK // 2
N_HOP = N_DEV - 1


def kernel(x, router_W, route_idx, expert_W, shared_W):
    def body(x_ref, router_ref, idx_ref, ew_ref, sw_ref, out_ref,
             gate_ref, ew_bf, sw_bf,
             r_rs_sbuf, r_rs_rbuf, r_ag_buf, r_own_buf,
             l_rs_sbuf, l_rs_rbuf, l_ag_buf, l_own_buf,
             r_rs_send, r_rs_recv, r_ag_send, r_ag_recv,
             l_rs_send, l_rs_recv, l_ag_send, l_ag_recv):
        my_pos = lax.axis_index("i")
        left = lax.rem(my_pos + N_DEV - 1, N_DEV)
        right = lax.rem(my_pos + 1, N_DEV)

        def r_rows(c):
            return pl.ds(c * CHUNK, HALF)

        def l_rows(c):
            return pl.ds(c * CHUNK + HALF, HALF)

        def cols(j):
            return slice(j * HCOL, (j + 1) * HCOL)

        barrier_sem = pltpu.get_barrier_semaphore()
        for nbr in (left, right):
            pl.semaphore_signal(
                barrier_sem, inc=1,
                device_id=(nbr,), device_id_type=pl.DeviceIdType.MESH,
            )
        pl.semaphore_wait(barrier_sem, 2)

        xv = x_ref[:, :]
        scores = jnp.dot(xv, router_ref[:, :], preferred_element_type=jnp.float32)
        s_max = jnp.max(scores, axis=-1, keepdims=True)
        probs = jnp.exp(scores - s_max)
        probs = probs / jnp.sum(probs, axis=-1, keepdims=True)
        route = idx_ref[:, :]
        iota = lax.broadcasted_iota(jnp.int32, (N_TOK, N_EXP), 1)
        onehot = (iota == route).astype(jnp.float32)
        p_tok = jnp.sum(probs * onehot, axis=-1, keepdims=True)
        for le in range(E_LOCAL):
            ge = my_pos * E_LOCAL + le
            gate_ref[:, le:le + 1] = jnp.where(route == ge, p_tok, 0.0)

        for le in range(E_LOCAL):
            ew_bf[le] = ew_ref[le].astype(jnp.bfloat16)
        sw_bf[:, :] = sw_ref[:, :].astype(jnp.bfloat16)

        def compute_half(row0, add_shared):
            x_c = x_ref[pl.ds(row0, HALF), :].astype(jnp.bfloat16)
            if add_shared:
                acc = jnp.dot(x_c, sw_bf[:, :],
                              preferred_element_type=jnp.float32)
            else:
                acc = jnp.zeros((HALF, H), jnp.float32)
            for le in range(E_LOCAL):
                y = jnp.dot(x_c, ew_bf[le],
                            preferred_element_type=jnp.float32)
                acc = acc + gate_ref[pl.ds(row0, HALF), le:le + 1] * y
            return acc[:, cols(0)], acc[:, cols(1)]

        def rs_desc(sbuf, rbuf, ssem, rsem, s, j, dev):
            return pltpu.make_async_remote_copy(
                src_ref=sbuf.at[s, :, cols(j)], dst_ref=rbuf.at[s, :, cols(j)],
                send_sem=ssem.at[s, j], recv_sem=rsem.at[s, j],
                device_id=(dev,), device_id_type=pl.DeviceIdType.MESH,
            )

        bfl = jnp.bfloat16
        f32 = jnp.float32

        c0 = my_pos
        acc_r0, acc_r1 = compute_half(c0 * CHUNK, False)
        r_rs_sbuf[0, :, cols(0)] = acc_r0.astype(bfl)
        rA = rs_desc(r_rs_sbuf, r_rs_rbuf, r_rs_send, r_rs_recv, 0, 0, right)
        rA.start()
        r_rs_sbuf[0, :, cols(1)] = acc_r1.astype(bfl)
        rB = rs_desc(r_rs_sbuf, r_rs_rbuf, r_rs_send, r_rs_recv, 0, 1, right)
        rB.start()
        acc_l0, acc_l1 = compute_half(c0 * CHUNK + HALF, False)
        l_rs_sbuf[0, :, cols(0)] = acc_l0.astype(bfl)
        lA = rs_desc(l_rs_sbuf, l_rs_rbuf, l_rs_send, l_rs_recv, 0, 0, left)
        lA.start()
        l_rs_sbuf[0, :, cols(1)] = acc_l1.astype(bfl)
        lB = rs_desc(l_rs_sbuf, l_rs_rbuf, l_rs_send, l_rs_recv, 0, 1, left)
        lB.start()

        for s in range(N_HOP):
            cr_r = lax.rem(my_pos + 2 * N_DEV - s - 1, N_DEV)
            cr_l = lax.rem(my_pos + s + 1, N_DEV)
            last = s == N_HOP - 1
            acc_r0, acc_r1 = compute_half(cr_r * CHUNK, last)
            acc_l0, acc_l1 = compute_half(cr_l * CHUNK + HALF, last)

            rA.wait()
            acc_r0 = acc_r0 + r_rs_rbuf[s, :, cols(0)].astype(f32)
            if not last:
                r_rs_sbuf[s + 1, :, cols(0)] = acc_r0.astype(bfl)
                rA = rs_desc(r_rs_sbuf, r_rs_rbuf, r_rs_send, r_rs_recv,
                             s + 1, 0, right)
                rA.start()
            else:
                r_own_buf[:, cols(0)] = acc_r0.astype(bfl)
            lA.wait()
            acc_l0 = acc_l0 + l_rs_rbuf[s, :, cols(0)].astype(f32)
            if not last:
                l_rs_sbuf[s + 1, :, cols(0)] = acc_l0.astype(bfl)
                lA = rs_desc(l_rs_sbuf, l_rs_rbuf, l_rs_send, l_rs_recv,
                             s + 1, 0, left)
                lA.start()
            else:
                l_own_buf[:, cols(0)] = acc_l0.astype(bfl)

            rB.wait()
            acc_r1 = acc_r1 + r_rs_rbuf[s, :, cols(1)].astype(f32)
            if not last:
                r_rs_sbuf[s + 1, :, cols(1)] = acc_r1.astype(bfl)
                rB = rs_desc(r_rs_sbuf, r_rs_rbuf, r_rs_send, r_rs_recv,
                             s + 1, 1, right)
                rB.start()
            else:
                r_own_buf[:, cols(1)] = acc_r1.astype(bfl)
            lB.wait()
            acc_l1 = acc_l1 + l_rs_rbuf[s, :, cols(1)].astype(f32)
            if not last:
                l_rs_sbuf[s + 1, :, cols(1)] = acc_l1.astype(bfl)
                lB = rs_desc(l_rs_sbuf, l_rs_rbuf, l_rs_send, l_rs_recv,
                             s + 1, 1, left)
                lB.start()
            else:
                l_own_buf[:, cols(1)] = acc_l1.astype(bfl)

            if last:
                own_r = cr_r
                own_l = cr_l
                out_ref[r_rows(own_r), cols(0)] = acc_r0
                out_ref[r_rows(own_r), cols(1)] = acc_r1
                out_ref[l_rows(own_l), cols(0)] = acc_l0
                out_ref[l_rows(own_l), cols(1)] = acc_l1

        def ag_desc(src, dst, ssem, rsem, s, j, dev):
            return pltpu.make_async_remote_copy(
                src_ref=src, dst_ref=dst,
                send_sem=ssem.at[s, j], recv_sem=rsem.at[s, j],
                device_id=(dev,), device_id_type=pl.DeviceIdType.MESH,
            )

        rA = ag_desc(r_own_buf.at[:, cols(0)], r_ag_buf.at[0, :, cols(0)],
                     r_ag_send, r_ag_recv, 0, 0, right)
        rA.start()
        lA = ag_desc(l_own_buf.at[:, cols(0)], l_ag_buf.at[0, :, cols(0)],
                     l_ag_send, l_ag_recv, 0, 0, left)
        lA.start()
        rB = ag_desc(r_own_buf.at[:, cols(1)], r_ag_buf.at[0, :, cols(1)],
                     r_ag_send, r_ag_recv, 0, 1, right)
        rB.start()
        lB = ag_desc(l_own_buf.at[:, cols(1)], l_ag_buf.at[0, :, cols(1)],
                     l_ag_send, l_ag_recv, 0, 1, left)
        lB.start()
        for s in range(N_HOP):
            c_r = lax.rem(my_pos + N_DEV - s, N_DEV)
            c_l = lax.rem(my_pos + s, N_DEV)
            rA.wait()
            if s < N_HOP - 1:
                nrA = ag_desc(r_ag_buf.at[s, :, cols(0)],
                              r_ag_buf.at[s + 1, :, cols(0)],
                              r_ag_send, r_ag_recv, s + 1, 0, right)
                nrA.start()
            out_ref[r_rows(c_r), cols(0)] = (
                r_ag_buf[s, :, cols(0)].astype(f32))
            lA.wait()
            if s < N_HOP - 1:
                nlA = ag_desc(l_ag_buf.at[s, :, cols(0)],
                              l_ag_buf.at[s + 1, :, cols(0)],
                              l_ag_send, l_ag_recv, s + 1, 0, left)
                nlA.start()
            out_ref[l_rows(c_l), cols(0)] = (
                l_ag_buf[s, :, cols(0)].astype(f32))
            rB.wait()
            if s < N_HOP - 1:
                nrB = ag_desc(r_ag_buf.at[s, :, cols(1)],
                              r_ag_buf.at[s + 1, :, cols(1)],
                              r_ag_send, r_ag_recv, s + 1, 1, right)
                nrB.start()
            out_ref[r_rows(c_r), cols(1)] = (
                r_ag_buf[s, :, cols(1)].astype(f32))
            lB.wait()
            if s < N_HOP - 1:
                nlB = ag_desc(l_ag_buf.at[s, :, cols(1)],
                              l_ag_buf.at[s + 1, :, cols(1)],
                              l_ag_send, l_ag_recv, s + 1, 1, left)
                nlB.start()
            out_ref[l_rows(c_l), cols(1)] = (
                l_ag_buf[s, :, cols(1)].astype(f32))
            if s < N_HOP - 1:
                rA, lA, rB, lB = nrA, nlA, nrB, nlB

    dma32 = pltpu.SemaphoreType.DMA((N_HOP, 2))
    bfl = jnp.bfloat16
    return pl.pallas_call(
        body,
        out_shape=jax.ShapeDtypeStruct((N_TOK, H), jnp.float32),
        in_specs=[pl.BlockSpec(memory_space=pltpu.VMEM)] * 5,
        out_specs=pl.BlockSpec(memory_space=pltpu.VMEM),
        scratch_shapes=[
            pltpu.VMEM((N_TOK, E_LOCAL), jnp.float32),
            pltpu.VMEM((E_LOCAL, D, H), bfl),
            pltpu.VMEM((D, H), bfl),
            pltpu.VMEM((N_HOP, HALF, H), bfl),
            pltpu.VMEM((N_HOP, HALF, H), bfl),
            pltpu.VMEM((N_HOP, HALF, H), bfl),
            pltpu.VMEM((HALF, H), bfl),
            pltpu.VMEM((N_HOP, HALF, H), bfl),
            pltpu.VMEM((N_HOP, HALF, H), bfl),
            pltpu.VMEM((N_HOP, HALF, H), bfl),
            pltpu.VMEM((HALF, H), bfl),
            dma32, dma32, dma32, dma32,
            dma32, dma32, dma32, dma32,
        ],
        compiler_params=pltpu.CompilerParams(
            collective_id=0, vmem_limit_bytes=100 * 1024 * 1024,
        ),
    )(x, router_W, route_idx, expert_W, shared_W)
